# initial kernel scaffold (unmeasured)
import jax
import jax.numpy as jnp
from jax import lax
from jax.experimental import pallas as pl
from jax.experimental.pallas import tpu as pltpu

N_DEV = 16


def kernel(x, w_mat, scale_x, scale_w):
    m, _ = x.shape
    _, n = w_mat.shape
    ch = m // N_DEV

    partial = jnp.dot(x, w_mat, preferred_element_type=jnp.int32)
    sc = (scale_x * scale_w).astype(jnp.float32).reshape(1, 1)

    def body(sc_ref, p_ref, out_ref,
             recv_buf, local_buf, send_buf, out_stage,
             rs_send_sems, rs_recv_sems, ag_send_sems, ag_recv_sems,
             credit_sem, local_sem, store_sem):
        me = lax.axis_index("i")
        left = lax.rem(me + N_DEV - 1, N_DEV)
        right = lax.rem(me + 1, N_DEV)

        for s in range(N_DEV - 1):
            if s >= 2:
                pl.semaphore_wait(credit_sem, 1)
            slot = s % 2
            send_c = lax.rem(me - s + N_DEV, N_DEV)
            src = p_ref.at[pl.ds(send_c * ch, ch), :] if s == 0 else send_buf
            rdma = pltpu.make_async_remote_copy(
                src_ref=src,
                dst_ref=recv_buf.at[slot],
                send_sem=rs_send_sems.at[s],
                recv_sem=rs_recv_sems.at[s],
                device_id=(right,),
                device_id_type=pl.DeviceIdType.MESH,
            )
            rdma.start()
            add_c = lax.rem(me - s - 1 + N_DEV, N_DEV)
            cp = pltpu.make_async_copy(
                p_ref.at[pl.ds(add_c * ch, ch), :], local_buf, local_sem)
            cp.start()
            rdma.wait()
            cp.wait()
            send_buf[...] = recv_buf[slot] + local_buf[...]
            if s + 2 <= N_DEV - 2:
                pl.semaphore_signal(credit_sem, inc=1, device_id=(left,),
                                    device_id_type=pl.DeviceIdType.MESH)

        out_stage[...] = send_buf[...].astype(jnp.float32) * sc_ref[0, 0]
        rc = lax.rem(me + 1, N_DEV)
        st = pltpu.make_async_copy(
            out_stage, out_ref.at[pl.ds(rc * ch, ch), :], store_sem)
        st.start()
        st.wait()

        for t in range(N_DEV - 1):
            c = lax.rem(me + 1 - t + N_DEV, N_DEV)
            rows = pl.ds(c * ch, ch)
            rdma = pltpu.make_async_remote_copy(
                src_ref=out_ref.at[rows, :],
                dst_ref=out_ref.at[rows, :],
                send_sem=ag_send_sems.at[t],
                recv_sem=ag_recv_sems.at[t],
                device_id=(right,),
                device_id_type=pl.DeviceIdType.MESH,
            )
            rdma.start()
            rdma.wait()

    return pl.pallas_call(
        body,
        out_shape=jax.ShapeDtypeStruct((m, n), jnp.float32),
        in_specs=[
            pl.BlockSpec(memory_space=pltpu.SMEM),
            pl.BlockSpec(memory_space=pl.ANY),
        ],
        out_specs=pl.BlockSpec(memory_space=pl.ANY),
        scratch_shapes=[
            pltpu.VMEM((2, ch, n), jnp.int32),
            pltpu.VMEM((ch, n), jnp.int32),
            pltpu.VMEM((ch, n), jnp.int32),
            pltpu.VMEM((ch, n), jnp.float32),
            pltpu.SemaphoreType.DMA((N_DEV - 1,)),
            pltpu.SemaphoreType.DMA((N_DEV - 1,)),
            pltpu.SemaphoreType.DMA((N_DEV - 1,)),
            pltpu.SemaphoreType.DMA((N_DEV - 1,)),
            pltpu.SemaphoreType.REGULAR,
            pltpu.SemaphoreType.DMA,
            pltpu.SemaphoreType.DMA,
        ],
    )(sc, partial)


# baseline (device time: 2918713 ns/iter reference)
import jax
import jax.numpy as jnp
from jax import lax
from jax.experimental import pallas as pl
from jax.experimental.pallas import tpu as pltpu

N_DEV = 16


def kernel(x, w_mat, scale_x, scale_w):
    m, _ = x.shape
    _, n = w_mat.shape
    ch = m // N_DEV

    partial = jnp.dot(x, w_mat, preferred_element_type=jnp.int32)
    sc = (scale_x * scale_w).astype(jnp.float32).reshape(1, 1)

    def body(sc_ref, p_ref, out_ref,
             recv_buf, local_buf, send_buf, out_stage,
             rs_send_sems, rs_recv_sems, ag_send_sems, ag_recv_sems,
             credit_sem, local_sem, store_sem):
        me = lax.axis_index("i")
        left = lax.rem(me + N_DEV - 1, N_DEV)
        right = lax.rem(me + 1, N_DEV)

        for s in range(N_DEV - 1):
            if s >= 2:
                pl.semaphore_wait(credit_sem, 1)
            slot = s % 2
            send_c = lax.rem(me - s + N_DEV, N_DEV)
            src = p_ref.at[pl.ds(send_c * ch, ch), :] if s == 0 else send_buf
            rdma = pltpu.make_async_remote_copy(
                src_ref=src,
                dst_ref=recv_buf.at[slot],
                send_sem=rs_send_sems.at[s],
                recv_sem=rs_recv_sems.at[s],
                device_id=(right,),
                device_id_type=pl.DeviceIdType.MESH,
            )
            rdma.start()
            add_c = lax.rem(me - s - 1 + N_DEV, N_DEV)
            cp = pltpu.make_async_copy(
                p_ref.at[pl.ds(add_c * ch, ch), :], local_buf, local_sem)
            cp.start()
            rdma.wait()
            cp.wait()
            send_buf[...] = recv_buf[slot] + local_buf[...]
            if s + 2 <= N_DEV - 2:
                pl.semaphore_signal(credit_sem, inc=1, device_id=(left,),
                                    device_id_type=pl.DeviceIdType.MESH)

        out_stage[...] = send_buf[...].astype(jnp.float32) * sc_ref[0, 0]
        rc = lax.rem(me + 1, N_DEV)
        st = pltpu.make_async_copy(
            out_stage, out_ref.at[pl.ds(rc * ch, ch), :], store_sem)
        st.start()
        st.wait()

        for t in range(N_DEV - 1):
            c = lax.rem(me + 1 - t + N_DEV, N_DEV)
            rows = pl.ds(c * ch, ch)
            rdma = pltpu.make_async_remote_copy(
                src_ref=out_ref.at[rows, :],
                dst_ref=out_ref.at[rows, :],
                send_sem=ag_send_sems.at[t],
                recv_sem=ag_recv_sems.at[t],
                device_id=(right,),
                device_id_type=pl.DeviceIdType.MESH,
            )
            rdma.start()
            rdma.wait()

    return pl.pallas_call(
        body,
        out_shape=jax.ShapeDtypeStruct((m, n), jnp.float32),
        in_specs=[
            pl.BlockSpec(memory_space=pltpu.SMEM),
            pl.BlockSpec(memory_space=pl.ANY),
        ],
        out_specs=pl.BlockSpec(memory_space=pl.ANY),
        scratch_shapes=[
            pltpu.VMEM((2, ch, n), jnp.int32),
            pltpu.VMEM((ch, n), jnp.int32),
            pltpu.VMEM((ch, n), jnp.int32),
            pltpu.VMEM((ch, n), jnp.float32),
            pltpu.SemaphoreType.DMA((N_DEV - 1,)),
            pltpu.SemaphoreType.DMA((N_DEV - 1,)),
            pltpu.SemaphoreType.DMA((N_DEV - 1,)),
            pltpu.SemaphoreType.DMA((N_DEV - 1,)),
            pltpu.SemaphoreType.REGULAR,
            pltpu.SemaphoreType.DMA,
            pltpu.SemaphoreType.DMA,
        ],
        compiler_params=pltpu.CompilerParams(
            vmem_limit_bytes=64 * 1024 * 1024,
        ),
    )(sc, partial)


# device time: 1603586 ns/iter; 1.8201x vs baseline; 1.8201x over previous
import jax
import jax.numpy as jnp
from jax import lax
from jax.experimental import pallas as pl
from jax.experimental.pallas import tpu as pltpu

N_DEV = 16


def kernel(x, w_mat, scale_x, scale_w):
    m, _ = x.shape
    _, n = w_mat.shape
    ch = m // N_DEV
    nh = n // 2

    partial = jnp.dot(x, w_mat, preferred_element_type=jnp.int32)
    sc = (scale_x * scale_w).astype(jnp.float32).reshape(1, 1)

    def body(sc_ref, p_ref, out_ref,
             recv_f, recv_b, local_f, local_b, send_f, send_b,
             stage_f, stage_b,
             rsf_ssem, rsf_rsem, rsb_ssem, rsb_rsem,
             agf_ssem, agf_rsem, agb_ssem, agb_rsem,
             credit_f, credit_b, lsem_f, lsem_b, stsem_f, stsem_b):
        me = lax.axis_index("i")
        left = lax.rem(me + N_DEV - 1, N_DEV)
        right = lax.rem(me + 1, N_DEV)
        COLS_F = pl.ds(0, nh)
        COLS_B = pl.ds(nh, nh)

        def remote(src, dst, ssem, rsem, dev):
            return pltpu.make_async_remote_copy(
                src_ref=src, dst_ref=dst, send_sem=ssem, recv_sem=rsem,
                device_id=(dev,), device_id_type=pl.DeviceIdType.MESH)

        for s in range(N_DEV - 1):
            if s >= 2:
                pl.semaphore_wait(credit_f, 1)
                pl.semaphore_wait(credit_b, 1)
            slot = s % 2
            cf = lax.rem(me - s + N_DEV, N_DEV)
            cb = lax.rem(me + s, N_DEV)
            src_f = p_ref.at[pl.ds(cf * ch, ch), COLS_F] if s == 0 else send_f
            src_b = p_ref.at[pl.ds(cb * ch, ch), COLS_B] if s == 0 else send_b
            rdma_f = remote(src_f, recv_f.at[slot],
                            rsf_ssem.at[s], rsf_rsem.at[s], right)
            rdma_b = remote(src_b, recv_b.at[slot],
                            rsb_ssem.at[s], rsb_rsem.at[s], left)
            rdma_f.start()
            rdma_b.start()
            af = lax.rem(me - s - 1 + N_DEV, N_DEV)
            ab = lax.rem(me + s + 1, N_DEV)
            cp_f = pltpu.make_async_copy(
                p_ref.at[pl.ds(af * ch, ch), COLS_F], local_f, lsem_f)
            cp_b = pltpu.make_async_copy(
                p_ref.at[pl.ds(ab * ch, ch), COLS_B], local_b, lsem_b)
            cp_f.start()
            cp_b.start()
            rdma_f.wait()
            cp_f.wait()
            send_f[...] = recv_f[slot] + local_f[...]
            rdma_b.wait()
            cp_b.wait()
            send_b[...] = recv_b[slot] + local_b[...]
            if s + 2 <= N_DEV - 2:
                pl.semaphore_signal(credit_f, inc=1, device_id=(left,),
                                    device_id_type=pl.DeviceIdType.MESH)
                pl.semaphore_signal(credit_b, inc=1, device_id=(right,),
                                    device_id_type=pl.DeviceIdType.MESH)

        stage_f[...] = send_f[...].astype(jnp.float32) * sc_ref[0, 0]
        stage_b[...] = send_b[...].astype(jnp.float32) * sc_ref[0, 0]
        rcf = lax.rem(me + 1, N_DEV)
        rcb = lax.rem(me + N_DEV - 1, N_DEV)
        st_f = pltpu.make_async_copy(
            stage_f, out_ref.at[pl.ds(rcf * ch, ch), COLS_F], stsem_f)
        st_b = pltpu.make_async_copy(
            stage_b, out_ref.at[pl.ds(rcb * ch, ch), COLS_B], stsem_b)
        st_f.start()
        st_b.start()
        st_f.wait()
        st_b.wait()

        for t in range(N_DEV - 1):
            cf = lax.rem(me + 1 - t + N_DEV, N_DEV)
            cb = lax.rem(me - 1 + t + N_DEV, N_DEV)
            rows_f = pl.ds(cf * ch, ch)
            rows_b = pl.ds(cb * ch, ch)
            rdma_f = remote(out_ref.at[rows_f, COLS_F],
                            out_ref.at[rows_f, COLS_F],
                            agf_ssem.at[t], agf_rsem.at[t], right)
            rdma_b = remote(out_ref.at[rows_b, COLS_B],
                            out_ref.at[rows_b, COLS_B],
                            agb_ssem.at[t], agb_rsem.at[t], left)
            rdma_f.start()
            rdma_b.start()
            rdma_f.wait()
            rdma_b.wait()

    nsem = N_DEV - 1
    return pl.pallas_call(
        body,
        out_shape=jax.ShapeDtypeStruct((m, n), jnp.float32),
        in_specs=[
            pl.BlockSpec(memory_space=pltpu.SMEM),
            pl.BlockSpec(memory_space=pl.ANY),
        ],
        out_specs=pl.BlockSpec(memory_space=pl.ANY),
        scratch_shapes=[
            pltpu.VMEM((2, ch, nh), jnp.int32),
            pltpu.VMEM((2, ch, nh), jnp.int32),
            pltpu.VMEM((ch, nh), jnp.int32),
            pltpu.VMEM((ch, nh), jnp.int32),
            pltpu.VMEM((ch, nh), jnp.int32),
            pltpu.VMEM((ch, nh), jnp.int32),
            pltpu.VMEM((ch, nh), jnp.float32),
            pltpu.VMEM((ch, nh), jnp.float32),
            pltpu.SemaphoreType.DMA((nsem,)),
            pltpu.SemaphoreType.DMA((nsem,)),
            pltpu.SemaphoreType.DMA((nsem,)),
            pltpu.SemaphoreType.DMA((nsem,)),
            pltpu.SemaphoreType.DMA((nsem,)),
            pltpu.SemaphoreType.DMA((nsem,)),
            pltpu.SemaphoreType.DMA((nsem,)),
            pltpu.SemaphoreType.DMA((nsem,)),
            pltpu.SemaphoreType.REGULAR,
            pltpu.SemaphoreType.REGULAR,
            pltpu.SemaphoreType.DMA,
            pltpu.SemaphoreType.DMA,
            pltpu.SemaphoreType.DMA,
            pltpu.SemaphoreType.DMA,
        ],
        compiler_params=pltpu.CompilerParams(
            vmem_limit_bytes=64 * 1024 * 1024,
        ),
    )(sc, partial)


# device time: 1500415 ns/iter; 1.9453x vs baseline; 1.0688x over previous
import jax
import jax.numpy as jnp
from jax import lax
from jax.experimental import pallas as pl
from jax.experimental.pallas import tpu as pltpu

N_DEV = 16
NSUB = 2


def kernel(x, w_mat, scale_x, scale_w):
    m, _ = x.shape
    _, n = w_mat.shape
    ch = m // N_DEV
    chh = ch // NSUB
    nh = n // 2

    partial = jnp.dot(x, w_mat, preferred_element_type=jnp.int32)
    sc = (scale_x * scale_w).astype(jnp.float32).reshape(1, 1)

    def body(sc_ref, p_ref, out_ref,
             recv_f, recv_b, send_f, send_b, loc_f, loc_b, stg_f, stg_b,
             rss_f, rsr_f, rss_b, rsr_b,
             ags_f, agr_f, ags_b, agr_b,
             lsem_f, lsem_b, stsem_f, stsem_b, cred_f, cred_b):
        me = lax.axis_index("i")
        left = lax.rem(me + N_DEV - 1, N_DEV)
        right = lax.rem(me + 1, N_DEV)

        class Dir:
            pass

        fwd = Dir()
        fwd.dev, fwd.sign, fwd.cred_to, fwd.col = right, -1, left, pl.ds(0, nh)
        fwd.recv, fwd.send, fwd.loc, fwd.stg = recv_f, send_f, loc_f, stg_f
        fwd.rss, fwd.rsr, fwd.ags, fwd.agr = rss_f, rsr_f, ags_f, agr_f
        fwd.lsem, fwd.stsem, fwd.cred = lsem_f, stsem_f, cred_f
        bwd = Dir()
        bwd.dev, bwd.sign, bwd.cred_to, bwd.col = left, 1, right, pl.ds(nh, nh)
        bwd.recv, bwd.send, bwd.loc, bwd.stg = recv_b, send_b, loc_b, stg_b
        bwd.rss, bwd.rsr, bwd.ags, bwd.agr = rss_b, rsr_b, ags_b, agr_b
        bwd.lsem, bwd.stsem, bwd.cred = lsem_b, stsem_b, cred_b
        dirs = (fwd, bwd)

        def chunk(k, sign):
            return lax.rem(me + sign * k + 2 * N_DEV, N_DEV)

        def rows(c, sub):
            return pl.ds(c * ch + sub * chh, chh)

        def remote(src, dst, ssem, rsem, dev):
            return pltpu.make_async_remote_copy(
                src_ref=src, dst_ref=dst, send_sem=ssem, recv_sem=rsem,
                device_id=(dev,), device_id_type=pl.DeviceIdType.MESH)

        rcur = [[None] * NSUB for _ in dirs]
        cps = [[None] * NSUB for _ in dirs]
        for di, D in enumerate(dirs):
            for sub in range(NSUB):
                r = remote(p_ref.at[rows(chunk(0, D.sign), sub), D.col],
                           D.recv.at[0, sub],
                           D.rss.at[0, sub], D.rsr.at[0, sub], D.dev)
                r.start()
                rcur[di][sub] = r
                cp = pltpu.make_async_copy(
                    p_ref.at[rows(chunk(1, D.sign), sub), D.col],
                    D.loc.at[0, sub], D.lsem.at[0, sub])
                cp.start()
                cps[di][sub] = cp

        for s in range(N_DEV - 1):
            slot = s % 2
            for sub in range(NSUB):
                for di, D in enumerate(dirs):
                    rcur[di][sub].wait()
                    cps[di][sub].wait()
                    D.send[sub] = D.recv[slot, sub] + D.loc[slot, sub]
                    if s < N_DEV - 2:
                        if sub == 0 and s + 1 >= 2:
                            pl.semaphore_wait(D.cred, 1)
                        r = remote(D.send.at[sub],
                                   D.recv.at[(s + 1) % 2, sub],
                                   D.rss.at[s + 1, sub],
                                   D.rsr.at[s + 1, sub], D.dev)
                        r.start()
                        rcur[di][sub] = r
            for di, D in enumerate(dirs):
                if s <= N_DEV - 4:
                    pl.semaphore_signal(D.cred, inc=1,
                                        device_id=(D.cred_to,),
                                        device_id_type=pl.DeviceIdType.MESH)
                if s < N_DEV - 2:
                    for sub in range(NSUB):
                        cp = pltpu.make_async_copy(
                            p_ref.at[rows(chunk(s + 2, D.sign), sub), D.col],
                            D.loc.at[(s + 1) % 2, sub],
                            D.lsem.at[(s + 1) % 2, sub])
                        cp.start()
                        cps[di][sub] = cp

        rag = [[None] * NSUB for _ in dirs]
        sts = [[None] * NSUB for _ in dirs]
        for di, D in enumerate(dirs):
            rc = chunk(-1, D.sign)
            for sub in range(NSUB):
                D.stg[sub] = D.send[sub].astype(jnp.float32) * sc_ref[0, 0]
                r = remote(D.stg.at[sub], out_ref.at[rows(rc, sub), D.col],
                           D.ags.at[0, sub], D.agr.at[0, sub], D.dev)
                r.start()
                rag[di][sub] = r
                st = pltpu.make_async_copy(
                    D.stg.at[sub], out_ref.at[rows(rc, sub), D.col],
                    D.stsem.at[sub])
                st.start()
                sts[di][sub] = st

        for t in range(N_DEV - 1):
            for sub in range(NSUB):
                for di, D in enumerate(dirs):
                    rag[di][sub].wait()
                    if t < N_DEV - 2:
                        cr = rows(chunk(t, D.sign), sub)
                        r = remote(out_ref.at[cr, D.col],
                                   out_ref.at[cr, D.col],
                                   D.ags.at[t + 1, sub],
                                   D.agr.at[t + 1, sub], D.dev)
                        r.start()
                        rag[di][sub] = r
        for di, D in enumerate(dirs):
            for sub in range(NSUB):
                sts[di][sub].wait()

    nsem = N_DEV - 1
    return pl.pallas_call(
        body,
        out_shape=jax.ShapeDtypeStruct((m, n), jnp.float32),
        in_specs=[
            pl.BlockSpec(memory_space=pltpu.SMEM),
            pl.BlockSpec(memory_space=pl.ANY),
        ],
        out_specs=pl.BlockSpec(memory_space=pl.ANY),
        scratch_shapes=[
            pltpu.VMEM((2, NSUB, chh, nh), jnp.int32),
            pltpu.VMEM((2, NSUB, chh, nh), jnp.int32),
            pltpu.VMEM((NSUB, chh, nh), jnp.int32),
            pltpu.VMEM((NSUB, chh, nh), jnp.int32),
            pltpu.VMEM((2, NSUB, chh, nh), jnp.int32),
            pltpu.VMEM((2, NSUB, chh, nh), jnp.int32),
            pltpu.VMEM((NSUB, chh, nh), jnp.float32),
            pltpu.VMEM((NSUB, chh, nh), jnp.float32),
            pltpu.SemaphoreType.DMA((nsem, NSUB)),
            pltpu.SemaphoreType.DMA((nsem, NSUB)),
            pltpu.SemaphoreType.DMA((nsem, NSUB)),
            pltpu.SemaphoreType.DMA((nsem, NSUB)),
            pltpu.SemaphoreType.DMA((nsem, NSUB)),
            pltpu.SemaphoreType.DMA((nsem, NSUB)),
            pltpu.SemaphoreType.DMA((nsem, NSUB)),
            pltpu.SemaphoreType.DMA((nsem, NSUB)),
            pltpu.SemaphoreType.DMA((2, NSUB)),
            pltpu.SemaphoreType.DMA((2, NSUB)),
            pltpu.SemaphoreType.DMA((NSUB,)),
            pltpu.SemaphoreType.DMA((NSUB,)),
            pltpu.SemaphoreType.REGULAR,
            pltpu.SemaphoreType.REGULAR,
        ],
        compiler_params=pltpu.CompilerParams(
            vmem_limit_bytes=64 * 1024 * 1024,
        ),
    )(sc, partial)
